# baseline (speedup 1.0000x reference)
import functools

import jax
import jax.numpy as jnp
from jax import lax
from jax.experimental import pallas as pl
from jax.experimental.pallas import tpu as pltpu
from jax.experimental.pallas import tpu_sc as plsc

B = 32
S = 512
H = 768
NW = 32
C = 64
NCH = S // C
EPS = 1e-12


def _gather_body(ids_h, word_h, tmp_h, idx_v, buf, sem0, sem1):
    cid = lax.axis_index("c")
    sid = lax.axis_index("s")
    w = sid * 2 + cid
    base = pl.multiple_of(w * S, S)

    pltpu.sync_copy(ids_h.at[pl.ds(base, S)], idx_v)

    sems = (sem0, sem1)

    def gather(c, k):
        pltpu.async_copy(
            word_h.at[idx_v.at[pl.ds(c * C, C)]], buf.at[k], sems[k])

    gather(0, 0)
    gather(1, 1)
    for c in range(NCH):
        k = c % 2
        pltpu.make_async_copy(
            word_h.at[idx_v.at[pl.ds(c * C, C)]], buf.at[k], sems[k]).wait()
        pltpu.sync_copy(buf.at[k], tmp_h.at[pl.ds(base + c * C, C)])
        if c + 2 < NCH:
            gather(c + 2, k)


def _sc_gather(ids_flat, word_table):
    mesh = plsc.VectorSubcoreMesh(core_axis_name="c", subcore_axis_name="s")
    f = pl.kernel(
        _gather_body,
        mesh=mesh,
        out_type=jax.ShapeDtypeStruct((B * S, H), jnp.float32),
        scratch_types=[
            pltpu.VMEM((S,), jnp.int32),
            pltpu.VMEM((2, C, H), jnp.float32),
            pltpu.SemaphoreType.DMA,
            pltpu.SemaphoreType.DMA,
        ],
    )
    return f(ids_flat, word_table)


_LNR = 256


def _ln_body(tmp_ref, pos_ref, gam_ref, bet_ref, out_ref):
    s_off = (pl.program_id(0) % (S // _LNR)) * _LNR
    x = tmp_ref[...] + pos_ref[pl.ds(s_off, _LNR), :]
    mean = jnp.mean(x, axis=-1, keepdims=True)
    q = jnp.mean(x * x, axis=-1, keepdims=True)
    rstd = lax.rsqrt(q - mean * mean + EPS)
    out_ref[...] = (x - mean) * rstd * gam_ref[...] + bet_ref[...]


def _tc_layernorm(tmp, pos_table, ln_gamma, ln_beta):
    return pl.pallas_call(
        _ln_body,
        grid=(B * S // _LNR,),
        in_specs=[
            pl.BlockSpec((_LNR, H), lambda b: (b, 0)),
            pl.BlockSpec((S, H), lambda b: (0, 0)),
            pl.BlockSpec((1, H), lambda b: (0, 0)),
            pl.BlockSpec((1, H), lambda b: (0, 0)),
        ],
        out_specs=pl.BlockSpec((_LNR, H), lambda b: (b, 0)),
        out_shape=jax.ShapeDtypeStruct((B * S, H), jnp.float32),
    )(tmp, pos_table, ln_gamma.reshape(1, H), ln_beta.reshape(1, H))


@jax.jit
def _emb_call(ids_flat, word_table, pos_table, ln_gamma, ln_beta):
    tmp = _sc_gather(ids_flat, word_table)
    out = _tc_layernorm(tmp, pos_table, ln_gamma, ln_beta)
    return out.reshape(B, S, H)


def kernel(input_ids, word_table, pos_table, ln_gamma, ln_beta):
    ids_flat = input_ids.astype(jnp.int32).reshape(B * S)
    return _emb_call(ids_flat, word_table, pos_table, ln_gamma, ln_beta)
